# baseline (speedup 1.0000x reference)
import functools

import jax
import jax.numpy as jnp
from jax import lax
from jax.experimental import pallas as pl
from jax.experimental.pallas import tpu as pltpu
from jax.experimental.pallas import tpu_sc as plsc

_N_SHARED = 2
_N_EXPERTS = 8
_TOP_K = 2
_D = 1024
_F = 4096
_T = 2048

_TILE = 256
_NPAD_R = _T * _TOP_K + _N_EXPERTS * _TILE
_NPAD = _NPAD_R + _N_SHARED * _T
_NTILES = _NPAD // _TILE
_NEG = -1e30


def _gating_body(x_ref, gw_ref, gb_ref, w_ref, e_ref):
    x = x_ref[...]
    logits = jnp.dot(x, gw_ref[...], preferred_element_type=jnp.float32)
    logits = logits + gb_ref[...]
    col = lax.broadcasted_iota(jnp.int32, logits.shape, 1)
    m1 = jnp.max(logits, axis=1, keepdims=True)
    a1 = jnp.argmax(logits, axis=1).astype(jnp.int32)
    masked = jnp.where(col == a1[:, None], _NEG, logits)
    m2 = jnp.max(masked, axis=1, keepdims=True)
    a2 = jnp.argmax(masked, axis=1).astype(jnp.int32)
    z = jnp.sum(jnp.exp(logits - m1), axis=1, keepdims=True)
    lse = m1 + jnp.log(z)
    w_ref[...] = jnp.concatenate([jnp.exp(m1 - lse), jnp.exp(m2 - lse)], axis=1)
    e_ref[...] = jnp.stack([a1, a2], axis=1)


def _gating(x, gate_W, gate_b):
    gw = jnp.zeros((_D, 128), jnp.float32).at[:, :_N_EXPERTS].set(gate_W)
    gb = jnp.full((1, 128), _NEG, jnp.float32).at[0, :_N_EXPERTS].set(gate_b)
    return pl.pallas_call(
        _gating_body,
        grid=(_T // 256,),
        in_specs=[
            pl.BlockSpec((256, _D), lambda i: (i, 0)),
            pl.BlockSpec((_D, 128), lambda i: (0, 0)),
            pl.BlockSpec((1, 128), lambda i: (0, 0)),
        ],
        out_specs=[
            pl.BlockSpec((256, _TOP_K), lambda i: (i, 0)),
            pl.BlockSpec((256, _TOP_K), lambda i: (i, 0)),
        ],
        out_shape=[
            jax.ShapeDtypeStruct((_T, _TOP_K), jnp.float32),
            jax.ShapeDtypeStruct((_T, _TOP_K), jnp.int32),
        ],
    )(x, gw, gb)


@functools.lru_cache(maxsize=None)
def _make_sc_gather(n_rows_table, n_rows_out):
    nw = 32
    per_w = n_rows_out // nw
    ch = 64
    n_ch = per_w // ch
    mesh = plsc.VectorSubcoreMesh(core_axis_name="c", subcore_axis_name="s")

    @functools.partial(
        pl.kernel,
        mesh=mesh,
        out_type=jax.ShapeDtypeStruct((n_rows_out, _D), jnp.float32),
        scratch_types=[
            pltpu.VMEM((n_ch, ch), jnp.int32),
            pltpu.VMEM((ch, _D), jnp.float32),
            pltpu.SemaphoreType.DMA,
        ],
    )
    def k(table_hbm, idx_hbm, out_hbm, idx_v, rows_v, sem):
        wid = lax.axis_index("s") * 2 + lax.axis_index("c")
        base = wid * per_w
        pltpu.sync_copy(idx_hbm.at[wid], idx_v)
        for ci in range(n_ch):
            pltpu.async_copy(table_hbm.at[idx_v.at[ci]], rows_v, sem).wait()
            pltpu.sync_copy(rows_v, out_hbm.at[pl.ds(base + ci * ch, ch)])

    def run(table, idx):
        idx3 = idx.reshape(nw, n_ch, ch)
        return k(table, idx3)

    return run


def _gather_x(table, idx):
    return _make_sc_gather(_T, _NPAD)(table, idx)


def _gather_y(table, idx):
    return _make_sc_gather(_NPAD, _T * _TOP_K)(table, idx)


def _ffn_body(eid_ref, x_ref, w1_ref, b1_ref, w2_ref, b2_ref, wp_ref, y_ref):
    del eid_ref
    x = x_ref[...].astype(jnp.bfloat16)
    h = jnp.dot(x, w1_ref[0], preferred_element_type=jnp.float32) + b1_ref[0]
    h = jax.nn.gelu(h)
    y = jnp.dot(h.astype(jnp.bfloat16), w2_ref[0],
                preferred_element_type=jnp.float32) + b2_ref[0]
    y_ref[...] = y * wp_ref[...]


def _grouped_ffn(x_sorted, W1, b1, W2, b2, w_pad, eids):
    grid_spec = pltpu.PrefetchScalarGridSpec(
        num_scalar_prefetch=1,
        grid=(_NTILES,),
        in_specs=[
            pl.BlockSpec((_TILE, _D), lambda i, eid: (i, 0)),
            pl.BlockSpec((1, _D, _F), lambda i, eid: (eid[i], 0, 0)),
            pl.BlockSpec((1, 1, _F), lambda i, eid: (eid[i], 0, 0)),
            pl.BlockSpec((1, _F, _D), lambda i, eid: (eid[i], 0, 0)),
            pl.BlockSpec((1, 1, _D), lambda i, eid: (eid[i], 0, 0)),
            pl.BlockSpec((_TILE, 1), lambda i, eid: (i, 0)),
        ],
        out_specs=pl.BlockSpec((_TILE, _D), lambda i, eid: (i, 0)),
    )
    return pl.pallas_call(
        _ffn_body,
        grid_spec=grid_spec,
        out_shape=jax.ShapeDtypeStruct((_NPAD, _D), jnp.float32),
    )(eids, x_sorted, W1, b1, W2, b2, w_pad)


def _combine_body(z0_ref, z1_ref, s0_ref, s1_ref, o_ref):
    o_ref[...] = z0_ref[...] + z1_ref[...] + s0_ref[...] + s1_ref[...]


def _combine(z, y):
    nt = _T // _TILE
    r = _NPAD_R // _TILE
    return pl.pallas_call(
        _combine_body,
        grid=(nt,),
        in_specs=[
            pl.BlockSpec((_TILE, _D), lambda i: (i, 0)),
            pl.BlockSpec((_TILE, _D), lambda i: (nt + i, 0)),
            pl.BlockSpec((_TILE, _D), lambda i: (r + i, 0)),
            pl.BlockSpec((_TILE, _D), lambda i: (r + nt + i, 0)),
        ],
        out_specs=pl.BlockSpec((_TILE, _D), lambda i: (i, 0)),
        out_shape=jax.ShapeDtypeStruct((_T, _D), jnp.float32),
    )(z, z, y, y)


def kernel(x, shared_W1, shared_b1, shared_W2, shared_b2,
           routed_W1, routed_b1, routed_W2, routed_b2, gate_W, gate_b):
    i32 = jnp.int32

    w, e = _gating(x, gate_W, gate_b)

    e_flat = e.reshape(-1)
    counts = jnp.bincount(e_flat, length=_N_EXPERTS).astype(i32)
    padded = ((counts + _TILE - 1) // _TILE) * _TILE
    poff = jnp.concatenate([jnp.zeros(1, i32), jnp.cumsum(padded)[:-1].astype(i32)])
    ustart = jnp.concatenate([jnp.zeros(1, i32), jnp.cumsum(counts)[:-1].astype(i32)])
    order = jnp.argsort(e_flat, stable=True).astype(i32)
    e_sorted = e_flat[order]
    ranks = jnp.arange(_T * _TOP_K, dtype=i32) - ustart[e_sorted]
    dest_sorted = poff[e_sorted] + ranks
    dest = jnp.zeros(_T * _TOP_K, i32).at[order].set(dest_sorted)

    row_ids = jnp.zeros(_NPAD_R, i32).at[dest].set(
        jnp.arange(_T * _TOP_K, dtype=i32) // _TOP_K)
    row_ids = jnp.concatenate(
        [row_ids, jnp.arange(_T, dtype=i32), jnp.arange(_T, dtype=i32)])

    w_pad = jnp.zeros(_NPAD_R, jnp.float32).at[dest].set(w.reshape(-1))
    w_pad = jnp.concatenate([w_pad, jnp.ones(_N_SHARED * _T, jnp.float32)])
    w_pad = w_pad.reshape(_NPAD, 1)

    ends = jnp.cumsum(padded).astype(i32)
    routed_tiles = jnp.arange(_NPAD_R // _TILE, dtype=i32) * _TILE
    tile_e = jnp.clip(jnp.searchsorted(ends, routed_tiles, side="right"),
                      0, _N_EXPERTS - 1).astype(i32)
    shared_tiles = jnp.repeat(
        jnp.arange(_N_EXPERTS, _N_EXPERTS + _N_SHARED, dtype=i32), _T // _TILE)
    eids = jnp.concatenate([tile_e, shared_tiles])

    W1 = jnp.concatenate([routed_W1, shared_W1]).astype(jnp.bfloat16)
    W2 = jnp.concatenate([routed_W2, shared_W2]).astype(jnp.bfloat16)
    b1 = jnp.concatenate([routed_b1, shared_b1]).reshape(
        _N_EXPERTS + _N_SHARED, 1, _F)
    b2 = jnp.concatenate([routed_b2, shared_b2]).reshape(
        _N_EXPERTS + _N_SHARED, 1, _D)

    x_sorted = _gather_x(x, row_ids)

    y = _grouped_ffn(x_sorted, W1, b1, W2, b2, w_pad, eids)

    idx_z = jnp.concatenate([dest[0::2], dest[1::2]])
    z = _gather_y(y, idx_z)

    return _combine(z, y)


# speedup vs baseline: 1.1807x; 1.1807x over previous
import functools

import jax
import jax.numpy as jnp
from jax import lax
from jax.experimental import pallas as pl
from jax.experimental.pallas import tpu as pltpu
from jax.experimental.pallas import tpu_sc as plsc

_N_SHARED = 2
_N_EXPERTS = 8
_TOP_K = 2
_D = 1024
_F = 4096
_T = 2048

_TILE = 256
_NPAD_R = _T * _TOP_K + _N_EXPERTS * _TILE
_NPAD = _NPAD_R + _N_SHARED * _T
_NTILES = _NPAD // _TILE
_NEG = -1e30


def _gating_body(x_ref, gw_ref, gb_ref, w_ref, e_ref):
    x = x_ref[...]
    logits = jnp.dot(x, gw_ref[...], preferred_element_type=jnp.float32)
    logits = logits + gb_ref[...]
    col = lax.broadcasted_iota(jnp.int32, logits.shape, 1)
    m1 = jnp.max(logits, axis=1, keepdims=True)
    a1 = jnp.argmax(logits, axis=1).astype(jnp.int32)
    masked = jnp.where(col == a1[:, None], _NEG, logits)
    m2 = jnp.max(masked, axis=1, keepdims=True)
    a2 = jnp.argmax(masked, axis=1).astype(jnp.int32)
    z = jnp.sum(jnp.exp(logits - m1), axis=1, keepdims=True)
    lse = m1 + jnp.log(z)
    w_ref[...] = jnp.concatenate([jnp.exp(m1 - lse), jnp.exp(m2 - lse)], axis=1)
    e_ref[...] = jnp.stack([a1, a2], axis=1)


def _gating(x, gate_W, gate_b):
    gw = jnp.zeros((_D, 128), jnp.float32).at[:, :_N_EXPERTS].set(gate_W)
    gb = jnp.full((1, 128), _NEG, jnp.float32).at[0, :_N_EXPERTS].set(gate_b)
    return pl.pallas_call(
        _gating_body,
        grid=(_T // 256,),
        in_specs=[
            pl.BlockSpec((256, _D), lambda i: (i, 0)),
            pl.BlockSpec((_D, 128), lambda i: (0, 0)),
            pl.BlockSpec((1, 128), lambda i: (0, 0)),
        ],
        out_specs=[
            pl.BlockSpec((256, _TOP_K), lambda i: (i, 0)),
            pl.BlockSpec((256, _TOP_K), lambda i: (i, 0)),
        ],
        out_shape=[
            jax.ShapeDtypeStruct((_T, _TOP_K), jnp.float32),
            jax.ShapeDtypeStruct((_T, _TOP_K), jnp.int32),
        ],
    )(x, gw, gb)


@functools.lru_cache(maxsize=None)
def _make_sc_gather(n_rows_table, n_rows_out):
    nw = 32
    per_w = n_rows_out // nw
    ch = 32
    n_ch = per_w // ch
    mesh = plsc.VectorSubcoreMesh(core_axis_name="c", subcore_axis_name="s")

    @functools.partial(
        pl.kernel,
        mesh=mesh,
        out_type=jax.ShapeDtypeStruct((n_rows_out, _D), jnp.float32),
        scratch_types=[
            pltpu.VMEM((n_ch, ch), jnp.int32),
            pltpu.VMEM((2, ch, _D), jnp.float32),
            pltpu.SemaphoreType.DMA,
            pltpu.SemaphoreType.DMA,
        ],
    )
    def k(table_hbm, idx_hbm, out_hbm, idx_v, rows_v, sem0, sem1):
        wid = lax.axis_index("s") * 2 + lax.axis_index("c")
        base = wid * per_w
        pltpu.sync_copy(idx_hbm.at[wid], idx_v)
        sems = (sem0, sem1)
        cps = [None] * n_ch
        cps[0] = pltpu.async_copy(table_hbm.at[idx_v.at[0]], rows_v.at[0], sems[0])
        for ci in range(n_ch):
            if ci + 1 < n_ch:
                cps[ci + 1] = pltpu.async_copy(
                    table_hbm.at[idx_v.at[ci + 1]], rows_v.at[(ci + 1) % 2],
                    sems[(ci + 1) % 2])
            cps[ci].wait()
            pltpu.sync_copy(rows_v.at[ci % 2], out_hbm.at[pl.ds(base + ci * ch, ch)])

    def run(table, idx):
        idx3 = idx.reshape(nw, n_ch, ch)
        return k(table, idx3)

    return run


def _gather_x(table, idx):
    return _make_sc_gather(_T, _NPAD_R)(table, idx)


def _gather_y(table, idx):
    return _make_sc_gather(_NPAD, _T * _TOP_K)(table, idx)


_N_RTILES = _NPAD_R // _TILE


def _ffn_body(eid_ref, xs_ref, xd_ref, w1_ref, b1_ref, w2_ref, b2_ref,
              wp_ref, y_ref):
    i = pl.program_id(0)
    is_shared = eid_ref[i] >= _N_EXPERTS
    x = jnp.where(is_shared, xd_ref[...], xs_ref[...]).astype(jnp.bfloat16)
    h = jnp.dot(x, w1_ref[0], preferred_element_type=jnp.float32) + b1_ref[0]
    h = jax.nn.gelu(h)
    y = jnp.dot(h.astype(jnp.bfloat16), w2_ref[0],
                preferred_element_type=jnp.float32) + b2_ref[0]
    y_ref[...] = y * wp_ref[...]


def _grouped_ffn(x_sorted, x, W1, b1, W2, b2, w_pad, eids):
    grid_spec = pltpu.PrefetchScalarGridSpec(
        num_scalar_prefetch=1,
        grid=(_NTILES,),
        in_specs=[
            pl.BlockSpec((_TILE, _D),
                         lambda i, eid: (jnp.where(i < _N_RTILES, i, 0), 0)),
            pl.BlockSpec((_TILE, _D),
                         lambda i, eid: (jnp.where(i < _N_RTILES, 0,
                                                   (i - _N_RTILES) % (_T // _TILE)), 0)),
            pl.BlockSpec((1, _D, _F), lambda i, eid: (eid[i], 0, 0)),
            pl.BlockSpec((1, 1, _F), lambda i, eid: (eid[i], 0, 0)),
            pl.BlockSpec((1, _F, _D), lambda i, eid: (eid[i], 0, 0)),
            pl.BlockSpec((1, 1, _D), lambda i, eid: (eid[i], 0, 0)),
            pl.BlockSpec((_TILE, 1), lambda i, eid: (i, 0)),
        ],
        out_specs=pl.BlockSpec((_TILE, _D), lambda i, eid: (i, 0)),
    )
    return pl.pallas_call(
        _ffn_body,
        grid_spec=grid_spec,
        out_shape=jax.ShapeDtypeStruct((_NPAD, _D), jnp.float32),
    )(eids, x_sorted, x, W1, b1, W2, b2, w_pad)


def _combine_body(z0_ref, z1_ref, s0_ref, s1_ref, o_ref):
    o_ref[...] = z0_ref[...] + z1_ref[...] + s0_ref[...] + s1_ref[...]


def _combine(z, y):
    nt = _T // _TILE
    r = _NPAD_R // _TILE
    return pl.pallas_call(
        _combine_body,
        grid=(nt,),
        in_specs=[
            pl.BlockSpec((_TILE, _D), lambda i: (i, 0)),
            pl.BlockSpec((_TILE, _D), lambda i: (nt + i, 0)),
            pl.BlockSpec((_TILE, _D), lambda i: (r + i, 0)),
            pl.BlockSpec((_TILE, _D), lambda i: (r + nt + i, 0)),
        ],
        out_specs=pl.BlockSpec((_TILE, _D), lambda i: (i, 0)),
        out_shape=jax.ShapeDtypeStruct((_T, _D), jnp.float32),
    )(z, z, y, y)


def kernel(x, shared_W1, shared_b1, shared_W2, shared_b2,
           routed_W1, routed_b1, routed_W2, routed_b2, gate_W, gate_b):
    i32 = jnp.int32

    w, e = _gating(x, gate_W, gate_b)

    e_flat = e.reshape(-1)
    counts = jnp.bincount(e_flat, length=_N_EXPERTS).astype(i32)
    padded = ((counts + _TILE - 1) // _TILE) * _TILE
    poff = jnp.concatenate([jnp.zeros(1, i32), jnp.cumsum(padded)[:-1].astype(i32)])
    ustart = jnp.concatenate([jnp.zeros(1, i32), jnp.cumsum(counts)[:-1].astype(i32)])
    order = jnp.argsort(e_flat, stable=True).astype(i32)
    e_sorted = e_flat[order]
    ranks = jnp.arange(_T * _TOP_K, dtype=i32) - ustart[e_sorted]
    dest_sorted = poff[e_sorted] + ranks
    dest = jnp.zeros(_T * _TOP_K, i32).at[order].set(dest_sorted)

    row_ids = (jnp.arange(_NPAD_R, dtype=i32) % _T).at[dest].set(
        jnp.arange(_T * _TOP_K, dtype=i32) // _TOP_K)

    w_pad = jnp.zeros(_NPAD_R, jnp.float32).at[dest].set(w.reshape(-1))
    w_pad = jnp.concatenate([w_pad, jnp.ones(_N_SHARED * _T, jnp.float32)])
    w_pad = w_pad.reshape(_NPAD, 1)

    ends = jnp.cumsum(padded).astype(i32)
    routed_tiles = jnp.arange(_NPAD_R // _TILE, dtype=i32) * _TILE
    tile_e = jnp.clip(jnp.searchsorted(ends, routed_tiles, side="right"),
                      0, _N_EXPERTS - 1).astype(i32)
    shared_tiles = jnp.repeat(
        jnp.arange(_N_EXPERTS, _N_EXPERTS + _N_SHARED, dtype=i32), _T // _TILE)
    eids = jnp.concatenate([tile_e, shared_tiles])

    W1 = jnp.concatenate([routed_W1, shared_W1]).astype(jnp.bfloat16)
    W2 = jnp.concatenate([routed_W2, shared_W2]).astype(jnp.bfloat16)
    b1 = jnp.concatenate([routed_b1, shared_b1]).reshape(
        _N_EXPERTS + _N_SHARED, 1, _F)
    b2 = jnp.concatenate([routed_b2, shared_b2]).reshape(
        _N_EXPERTS + _N_SHARED, 1, _D)

    x_sorted = _gather_x(x, row_ids)

    y = _grouped_ffn(x_sorted, x, W1, b1, W2, b2, w_pad, eids)

    idx_z = jnp.concatenate([dest[0::2], dest[1::2]])
    z = _gather_y(y, idx_z)

    return _combine(z, y)
